# baseline (speedup 1.0000x reference)
import functools

import jax
import jax.numpy as jnp
from jax import lax
from jax.experimental import pallas as pl
from jax.experimental.pallas import tpu as pltpu
from jax.experimental.pallas import tpu_sc as plsc

_MAX_NODE_COUNT = 1000000
_EMBED_DIM = 64
_MAX_NORM = 1.0
_BATCH = 16384

_NC = 2
_NS = 16
_NW = _NC * _NS
_B_PER_W = _BATCH // _NW
_CHUNK = 128
_N_CHUNKS = _B_PER_W // _CHUNK


def _rsqrt_newton(x):
    i = lax.bitcast_convert_type(x, jnp.int32)
    i = jnp.int32(0x5F3759DF) - (i >> 1)
    y = lax.bitcast_convert_type(i, jnp.float32)
    for _ in range(3):
        y = y * (1.5 - 0.5 * x * y * y)
    return y


def _make_kernel():
    mesh = plsc.VectorSubcoreMesh(core_axis_name="c", subcore_axis_name="s")

    @functools.partial(
        pl.kernel,
        mesh=mesh,
        compiler_params=pltpu.CompilerParams(
            needs_layout_passes=False, use_tc_tiling_on_sc=False),
        out_type=jax.ShapeDtypeStruct((_BATCH, _EMBED_DIM), jnp.float32),
        scratch_types=[
            pltpu.VMEM((_B_PER_W,), jnp.int32),
            pltpu.VMEM((_B_PER_W, _EMBED_DIM), jnp.float32),
            pltpu.SemaphoreType.DMA,
        ],
    )
    def emb_kernel(ids_hbm, table_hbm, out_hbm, idx_v, rows_v, sem):
        wid = lax.axis_index("s") * _NC + lax.axis_index("c")
        base = wid * _B_PER_W

        pltpu.sync_copy(ids_hbm.at[pl.ds(base, _B_PER_W)], idx_v)

        copies = []
        for c in range(_N_CHUNKS):
            copies.append(
                pltpu.async_copy(
                    table_hbm.at[idx_v.at[pl.ds(c * _CHUNK, _CHUNK)]],
                    rows_v.at[pl.ds(c * _CHUNK, _CHUNK)],
                    sem,
                )
            )
        for cp in copies:
            cp.wait()

        def row_body(r, _):
            v0 = rows_v[r, pl.ds(0, 16)]
            v1 = rows_v[r, pl.ds(16, 16)]
            v2 = rows_v[r, pl.ds(32, 16)]
            v3 = rows_v[r, pl.ds(48, 16)]
            acc = v0 * v0 + v1 * v1 + v2 * v2 + v3 * v3
            ssq = jnp.sum(acc)
            scale = jnp.where(ssq > _MAX_NORM * _MAX_NORM,
                              _rsqrt_newton(ssq) * _MAX_NORM,
                              jnp.float32(1.0))
            rows_v[r, pl.ds(0, 16)] = v0 * scale
            rows_v[r, pl.ds(16, 16)] = v1 * scale
            rows_v[r, pl.ds(32, 16)] = v2 * scale
            rows_v[r, pl.ds(48, 16)] = v3 * scale
            return 0

        lax.fori_loop(0, _B_PER_W, row_body, 0)

        pltpu.sync_copy(rows_v, out_hbm.at[pl.ds(base, _B_PER_W)])

    return emb_kernel


_emb_kernel = _make_kernel()


@jax.jit
def kernel(node_ids, table):
    return _emb_kernel(node_ids, table)
